# Initial kernel scaffold; baseline (speedup 1.0000x reference)
import jax
import jax.numpy as jnp
from jax.experimental import pallas as pl


def kernel(boxes, scores):
    raise NotImplementedError("write your pallas kernel here")


# speedup vs baseline: 36.9758x; 36.9758x over previous
import functools

import jax
import jax.numpy as jnp
from jax import lax
from jax.experimental import pallas as pl
from jax.experimental.pallas import tpu as pltpu

N_REAL = 5000
B = 128
NB = 40
NP = NB * B
PROB_THR = 0.5
IOU_THR = 0.5


def _nms_body(boxes_ref, s_row_ref, s_col_ref, out_ref,
              rank_s, sd_col_s, sd_row_s, geo_s, keep_s):
    f32 = jnp.float32
    col_np = lax.broadcasted_iota(jnp.int32, (1, NP), 1)

    def rank_blk(a, _):
        s_blk = s_col_ref[pl.ds(a * B, B), :]
        i_glob = (a * B
                  + lax.broadcasted_iota(jnp.int32, (B, 1), 0))
        s_all = s_row_ref[...]
        gt = (s_all > s_blk)
        tie = (s_all == s_blk) & (col_np < i_glob)
        cnt = jnp.sum((gt | tie).astype(jnp.int32), axis=1,
                      keepdims=True)
        rank_s[pl.ds(a * B, B), :] = cnt
        return 0

    lax.fori_loop(0, NB, rank_blk, 0, unroll=False)

    data = jnp.concatenate([boxes_ref[...], s_col_ref[...]], axis=1)
    rank_col = rank_s[...]

    def perm_blk(k, _):
        cols = k * B + lax.broadcasted_iota(jnp.int32, (1, B), 1)
        p = (rank_col == cols).astype(f32)
        blk_col = lax.dot_general(p, data, (((0,), (0,)), ((), ())),
                                  preferred_element_type=f32,
                                  precision=lax.Precision.HIGHEST)
        sd_col_s[pl.ds(k * B, B), :] = blk_col
        blk_row = lax.dot_general(data, p, (((0,), (0,)), ((), ())),
                                  preferred_element_type=f32,
                                  precision=lax.Precision.HIGHEST)
        sd_row_s[0:5, pl.ds(k * B, B)] = blk_row
        return 0

    lax.fori_loop(0, NB, perm_blk, 0, unroll=False)

    x0 = sd_row_s[0:1, :]
    y0 = sd_row_s[1:2, :]
    x1 = sd_row_s[2:3, :]
    y1 = sd_row_s[3:4, :]
    lox = jnp.minimum(x0, x1)
    loy = jnp.minimum(y0, y1)
    hix = jnp.maximum(x0, x1)
    hiy = jnp.maximum(y0, y1)
    geo_s[0:1, :] = lox
    geo_s[1:2, :] = loy
    geo_s[2:3, :] = hix
    geo_s[3:4, :] = hiy
    geo_s[4:5, :] = (hix - lox) * (hiy - loy)

    keep_s[...] = (sd_row_s[4:5, :] > PROB_THR).astype(f32)

    ut_mask = (lax.broadcasted_iota(jnp.int32, (B, B), 0)
               < lax.broadcasted_iota(jnp.int32, (B, B), 1)).astype(f32)
    iota_b = lax.broadcasted_iota(jnp.int32, (1, B), 1)

    def nms_blk(k, _):
        cb = sd_col_s[pl.ds(k * B, B), :]
        bx0 = cb[:, 0:1]
        by0 = cb[:, 1:2]
        bx1 = cb[:, 2:3]
        by1 = cb[:, 3:4]
        lox_b = jnp.minimum(bx0, bx1)
        loy_b = jnp.minimum(by0, by1)
        hix_b = jnp.maximum(bx0, bx1)
        hiy_b = jnp.maximum(by0, by1)
        area_b = (hix_b - lox_b) * (hiy_b - loy_b)

        def iou_vs(lox_r, loy_r, hix_r, hiy_r, area_r):
            iw = jnp.clip(jnp.minimum(hix_b, hix_r)
                          - jnp.maximum(lox_b, lox_r), 0.0, None)
            ih = jnp.clip(jnp.minimum(hiy_b, hiy_r)
                          - jnp.maximum(loy_b, loy_r), 0.0, None)
            inter = iw * ih
            union = area_b + area_r - inter
            return inter / (union + 1e-8)

        sl = pl.ds(k * B, B)
        s_bb = (iou_vs(geo_s[0:1, sl], geo_s[1:2, sl],
                       geo_s[2:3, sl], geo_s[3:4, sl],
                       geo_s[4:5, sl]) > IOU_THR).astype(f32)
        s_ut = s_bb * ut_mask

        valid = keep_s[0:1, sl]

        def fix_cond(c):
            return c[1] > 0

        def fix_body(c):
            kb, _ = c
            supp = lax.dot_general(kb, s_ut, (((1,), (0,)), ((), ())),
                                   preferred_element_type=f32,
                                  precision=lax.Precision.HIGHEST)
            new = valid * (supp < 0.5).astype(f32)
            changed = jnp.sum((new != kb).astype(jnp.int32))
            return (new, changed)

        keep_blk, _ = lax.while_loop(fix_cond, fix_body,
                                     (valid, jnp.int32(1)))
        keep_s[0:1, sl] = keep_blk

        s_all = (iou_vs(geo_s[0:1, :], geo_s[1:2, :],
                        geo_s[2:3, :], geo_s[3:4, :],
                        geo_s[4:5, :]) > IOU_THR).astype(f32)
        supp_all = lax.dot_general(keep_blk, s_all,
                                   (((1,), (0,)), ((), ())),
                                   preferred_element_type=f32,
                                  precision=lax.Precision.HIGHEST)
        later = (col_np >= (k + 1) * B) & (supp_all > 0.5)
        keep_s[...] = keep_s[...] * (1.0 - later.astype(f32))
        return 0

    lax.fori_loop(0, NB, nms_blk, 0, unroll=False)

    m = keep_s[...]
    out_ref[0:5, :] = sd_row_s[0:5, :] * m
    out_ref[5:8, :] = jnp.zeros((3, NP), f32)


@jax.jit
def kernel(boxes, scores):
    pad = NP - N_REAL
    boxes_p = jnp.pad(boxes.astype(jnp.float32), ((0, pad), (0, 0)))
    scores_p = jnp.pad(scores.astype(jnp.float32), (0, pad),
                       constant_values=-1.0)
    s_row = scores_p.reshape(1, NP)
    s_col = scores_p.reshape(NP, 1)

    out_row = pl.pallas_call(
        _nms_body,
        out_shape=jax.ShapeDtypeStruct((8, NP), jnp.float32),
        scratch_shapes=[
            pltpu.VMEM((NP, 1), jnp.int32),
            pltpu.VMEM((NP, 5), jnp.float32),
            pltpu.VMEM((8, NP), jnp.float32),
            pltpu.VMEM((8, NP), jnp.float32),
            pltpu.VMEM((1, NP), jnp.float32),
        ],
    )(boxes_p, s_row, s_col)

    return out_row[0:5, :N_REAL].T


# speedup vs baseline: 71.1649x; 1.9246x over previous
import functools

import jax
import jax.numpy as jnp
from jax import lax
from jax.experimental import pallas as pl
from jax.experimental.pallas import tpu as pltpu

N_REAL = 5000
B = 128
NB = 40
NP = NB * B
PROB_THR = 0.5
IOU_THR = 0.5


def _nms_body(boxes_ref, s_row_ref, s_col_ref, out_ref,
              rank_s, sd_col_s, sd_row_s, geo_s, keep_s):
    f32 = jnp.float32
    col_np = lax.broadcasted_iota(jnp.int32, (1, NP), 1)

    n_valid = jnp.sum((s_row_ref[...] > PROB_THR).astype(jnp.int32))
    nba = lax.div(n_valid + (B - 1), B)

    sd_row_s[...] = jnp.zeros((8, NP), f32)

    def rank_blk(a, _):
        s_blk = s_col_ref[pl.ds(a * B, B), :]
        i_glob = (a * B
                  + lax.broadcasted_iota(jnp.int32, (B, 1), 0))
        s_all = s_row_ref[...]
        gt = (s_all > s_blk)
        tie = (s_all == s_blk) & (col_np < i_glob)
        cnt = jnp.sum((gt | tie).astype(jnp.int32), axis=1,
                      keepdims=True)
        rank_s[pl.ds(a * B, B), :] = cnt
        return 0

    lax.fori_loop(0, NB, rank_blk, 0, unroll=False)

    data = jnp.concatenate([boxes_ref[...], s_col_ref[...]], axis=1)
    rank_col = rank_s[...]

    def perm_blk(k, _):
        cols = k * B + lax.broadcasted_iota(jnp.int32, (1, B), 1)
        p = (rank_col == cols).astype(f32)
        blk_col = lax.dot_general(p, data, (((0,), (0,)), ((), ())),
                                  preferred_element_type=f32,
                                  precision=lax.Precision.HIGHEST)
        sd_col_s[pl.ds(k * B, B), :] = blk_col
        blk_row = lax.dot_general(data, p, (((0,), (0,)), ((), ())),
                                  preferred_element_type=f32,
                                  precision=lax.Precision.HIGHEST)
        sd_row_s[0:5, pl.ds(k * B, B)] = blk_row
        return 0

    lax.fori_loop(0, nba, perm_blk, 0, unroll=False)

    x0 = sd_row_s[0:1, :]
    y0 = sd_row_s[1:2, :]
    x1 = sd_row_s[2:3, :]
    y1 = sd_row_s[3:4, :]
    lox = jnp.minimum(x0, x1)
    loy = jnp.minimum(y0, y1)
    hix = jnp.maximum(x0, x1)
    hiy = jnp.maximum(y0, y1)
    geo_s[0:1, :] = lox
    geo_s[1:2, :] = loy
    geo_s[2:3, :] = hix
    geo_s[3:4, :] = hiy
    geo_s[4:5, :] = (hix - lox) * (hiy - loy)

    keep_s[...] = (sd_row_s[4:5, :] > PROB_THR).astype(f32)

    ut_mask = (lax.broadcasted_iota(jnp.int32, (B, B), 0)
               < lax.broadcasted_iota(jnp.int32, (B, B), 1)).astype(f32)
    iota_b = lax.broadcasted_iota(jnp.int32, (1, B), 1)

    def nms_blk(k, _):
        cb = sd_col_s[pl.ds(k * B, B), :]
        bx0 = cb[:, 0:1]
        by0 = cb[:, 1:2]
        bx1 = cb[:, 2:3]
        by1 = cb[:, 3:4]
        lox_b = jnp.minimum(bx0, bx1)
        loy_b = jnp.minimum(by0, by1)
        hix_b = jnp.maximum(bx0, bx1)
        hiy_b = jnp.maximum(by0, by1)
        area_b = (hix_b - lox_b) * (hiy_b - loy_b)

        def iou_vs(lox_r, loy_r, hix_r, hiy_r, area_r):
            iw = jnp.clip(jnp.minimum(hix_b, hix_r)
                          - jnp.maximum(lox_b, lox_r), 0.0, None)
            ih = jnp.clip(jnp.minimum(hiy_b, hiy_r)
                          - jnp.maximum(loy_b, loy_r), 0.0, None)
            inter = iw * ih
            union = area_b + area_r - inter
            return inter / (union + 1e-8)

        sl = pl.ds(k * B, B)
        s_bb = (iou_vs(geo_s[0:1, sl], geo_s[1:2, sl],
                       geo_s[2:3, sl], geo_s[3:4, sl],
                       geo_s[4:5, sl]) > IOU_THR).astype(f32)
        s_ut = s_bb * ut_mask

        valid = keep_s[0:1, sl]

        def fix_cond(c):
            return c[1] > 0

        def fix_body(c):
            kb, _ = c
            supp = lax.dot_general(kb, s_ut, (((1,), (0,)), ((), ())),
                                   preferred_element_type=f32)
            new = valid * (supp < 0.5).astype(f32)
            changed = jnp.sum((new != kb).astype(jnp.int32))
            return (new, changed)

        keep_blk, _ = lax.while_loop(fix_cond, fix_body,
                                     (valid, jnp.int32(1)))
        keep_s[0:1, sl] = keep_blk

        s_all = (iou_vs(geo_s[0:1, :], geo_s[1:2, :],
                        geo_s[2:3, :], geo_s[3:4, :],
                        geo_s[4:5, :]) > IOU_THR).astype(f32)
        supp_all = lax.dot_general(keep_blk, s_all,
                                   (((1,), (0,)), ((), ())),
                                   preferred_element_type=f32)
        later = (col_np >= (k + 1) * B) & (supp_all > 0.5)
        keep_s[...] = keep_s[...] * (1.0 - later.astype(f32))
        return 0

    lax.fori_loop(0, nba, nms_blk, 0, unroll=False)

    m = keep_s[...]
    out_ref[0:5, :] = sd_row_s[0:5, :] * m
    out_ref[5:8, :] = jnp.zeros((3, NP), f32)


@jax.jit
def kernel(boxes, scores):
    pad = NP - N_REAL
    boxes_p = jnp.pad(boxes.astype(jnp.float32), ((0, pad), (0, 0)))
    scores_p = jnp.pad(scores.astype(jnp.float32), (0, pad),
                       constant_values=-1.0)
    s_row = scores_p.reshape(1, NP)
    s_col = scores_p.reshape(NP, 1)

    out_row = pl.pallas_call(
        _nms_body,
        out_shape=jax.ShapeDtypeStruct((8, NP), jnp.float32),
        scratch_shapes=[
            pltpu.VMEM((NP, 1), jnp.int32),
            pltpu.VMEM((NP, 5), jnp.float32),
            pltpu.VMEM((8, NP), jnp.float32),
            pltpu.VMEM((8, NP), jnp.float32),
            pltpu.VMEM((1, NP), jnp.float32),
        ],
    )(boxes_p, s_row, s_col)

    return out_row[0:5, :N_REAL].T
